# baseline (speedup 1.0000x reference)
import jax
import jax.numpy as jnp
from jax.experimental import pallas as pl
from jax.experimental.pallas import tpu as pltpu

HID = 1024
D2 = 2 * HID
E = 16
K = 2
BT = 4096
CHUNK = 512
N_CHUNK = BT // CHUNK
TOK_BLK = 512
N_TOK_BLK = BT // TOK_BLK
NEG_INF = float("-inf")


def _scores_kernel(x_ref, g_ref, out_ref):
    out_ref[...] = jnp.dot(x_ref[...], g_ref[...],
                           preferred_element_type=jnp.float32)


def _topk_kernel(s_ref, ti_ref, ts_ref):
    s = s_ref[...]
    iota = jax.lax.broadcasted_iota(jnp.int32, s.shape, 0)
    m1 = jnp.max(s, axis=0, keepdims=True)
    i1 = jnp.min(jnp.where(s == m1, iota, BT), axis=0, keepdims=True)
    s2 = jnp.where(iota == i1, NEG_INF, s)
    m2 = jnp.max(s2, axis=0, keepdims=True)
    i2 = jnp.min(jnp.where(s2 == m2, iota, BT), axis=0, keepdims=True)
    ti_ref[...] = jnp.concatenate([i1, i2], axis=0)
    ts_ref[...] = jnp.concatenate([m1, m2], axis=0)


def _gather_kernel(fg_ref, x_any, out_ref, sem):
    def start(j, _):
        pltpu.make_async_copy(
            x_any.at[pl.ds(fg_ref[j], 1)], out_ref.at[pl.ds(j, 1)],
            sem).start()
        return 0
    jax.lax.fori_loop(0, E * K, start, 0, unroll=True)

    def drain(j, _):
        pltpu.make_async_copy(
            x_any.at[pl.ds(fg_ref[j], 1)], out_ref.at[pl.ds(j, 1)],
            sem).wait()
        return 0
    jax.lax.fori_loop(0, E * K, drain, 0, unroll=True)


def _half_bits_to_f32(bits):
    sign = (bits & 0x8000) << 16
    expmant = (bits & 0x7FFF) << 13
    return jax.lax.bitcast_convert_type(sign | expmant, jnp.float32) * jnp.float32(
        5.192296858534828e33)


def _expert_kernel(xc_ref, w_ref, yr_ref, yi_ref):
    e = pl.program_id(0)
    wv = w_ref[...]
    wr = _half_bits_to_f32(wv & 0xFFFF).astype(jnp.bfloat16)
    wi = _half_bits_to_f32((wv >> 16) & 0xFFFF).astype(jnp.bfloat16)
    wcat = jnp.concatenate([wr, wi], axis=1)
    yy = jnp.dot(xc_ref[...], wcat, preferred_element_type=jnp.float32)
    m = E * K
    yr = yy[:m, :HID] - yy[m:, HID:]
    yi = yy[:m, HID:] + yy[m:, :HID]
    row = jax.lax.broadcasted_iota(jnp.int32, yr.shape, 0)
    own = row // K == e
    yr_ref[...] = jnp.where(own, yr, yr_ref[...])
    yi_ref[...] = jnp.where(own, yi, yi_ref[...])


def _combine_kernel(fs_ref, ts_ref, yw_ref, bias_ref, out_ref, cnt_ref):
    blk = pl.program_id(0)
    tok = jax.lax.broadcasted_iota(jnp.int32, (TOK_BLK, E * K), 0) + blk * TOK_BLK
    fs_row = fs_ref[0]
    one_hot = jnp.where(tok == fs_row, 1.0, 0.0).astype(jnp.float32)
    weight_oh = one_hot * ts_ref[0]
    out_sum = jnp.dot(weight_oh, yw_ref[...],
                      preferred_element_type=jnp.float32,
                      precision=jax.lax.Precision.HIGHEST)
    cnt = jnp.sum(one_hot, axis=1, keepdims=True)
    agg = out_sum / jnp.maximum(cnt, 1.0)
    z = agg + bias_ref[...]
    out_ref[...] = 0.5 * z * (1.0 + jax.lax.erf(z * 0.7071067811865476))
    cnt_ref[...] = cnt


@jax.jit
def kernel(x, gate_weights, experts_weight_real, act_bias):
    x_flat = x.reshape(BT, D2)

    scores = pl.pallas_call(
        _scores_kernel,
        grid=(N_CHUNK,),
        in_specs=[
            pl.BlockSpec((CHUNK, D2), lambda i: (i, 0)),
            pl.BlockSpec((D2, E), lambda i: (0, 0)),
        ],
        out_specs=pl.BlockSpec((CHUNK, E), lambda i: (i, 0)),
        out_shape=jax.ShapeDtypeStruct((BT, E), jnp.float32),
    )(x_flat, gate_weights)

    ti_t, ts_t = pl.pallas_call(
        _topk_kernel,
        out_shape=(
            jax.ShapeDtypeStruct((K, E), jnp.int32),
            jax.ShapeDtypeStruct((K, E), jnp.float32),
        ),
    )(scores)

    topk_indices = ti_t.T
    topk_scores = ts_t.T
    flat_gather = topk_indices.reshape(-1)
    flat_scatter = ti_t.reshape(-1)

    xb = pl.pallas_call(
        _gather_kernel,
        in_specs=[
            pl.BlockSpec(memory_space=pltpu.SMEM),
            pl.BlockSpec(memory_space=pl.ANY),
        ],
        out_specs=pl.BlockSpec(memory_space=pl.ANY),
        out_shape=jax.ShapeDtypeStruct((E * K, HID, 2), jnp.float32),
        scratch_shapes=[pltpu.SemaphoreType.DMA],
    )(flat_gather, x)

    xr = xb[..., 0].astype(jnp.float16).astype(jnp.bfloat16)
    xi = xb[..., 1].astype(jnp.float16).astype(jnp.bfloat16)
    xc = jnp.concatenate([xr, xi], axis=0)
    w_pairs = jax.lax.bitcast_convert_type(
        experts_weight_real, jnp.int32).reshape(E * HID, HID)

    yr_all, yi_all = pl.pallas_call(
        _expert_kernel,
        grid=(E,),
        in_specs=[
            pl.BlockSpec((2 * E * K, HID), lambda e: (0, 0)),
            pl.BlockSpec((HID, HID), lambda e: (e, 0)),
        ],
        out_specs=(
            pl.BlockSpec((E * K, HID), lambda e: (0, 0)),
            pl.BlockSpec((E * K, HID), lambda e: (0, 0)),
        ),
        out_shape=(
            jax.ShapeDtypeStruct((E * K, HID), jnp.float32),
            jax.ShapeDtypeStruct((E * K, HID), jnp.float32),
        ),
    )(xc, w_pairs)
    y_all = jnp.stack([yr_all, yi_all], axis=-1).reshape(E * K, D2)

    fs3 = flat_scatter.reshape(1, 1, E * K)
    ts3 = topk_scores.reshape(1, 1, E * K)
    bias2 = jnp.repeat(act_bias, 2).reshape(1, D2)

    res2d, cnt = pl.pallas_call(
        _combine_kernel,
        grid=(N_TOK_BLK,),
        in_specs=[
            pl.BlockSpec((1, 1, E * K), lambda i: (0, 0, 0)),
            pl.BlockSpec((1, 1, E * K), lambda i: (0, 0, 0)),
            pl.BlockSpec((E * K, D2), lambda i: (0, 0)),
            pl.BlockSpec((1, D2), lambda i: (0, 0)),
        ],
        out_specs=(
            pl.BlockSpec((TOK_BLK, D2), lambda i: (i, 0)),
            pl.BlockSpec((TOK_BLK, 1), lambda i: (i, 0)),
        ),
        out_shape=(
            jax.ShapeDtypeStruct((BT, D2), jnp.float32),
            jax.ShapeDtypeStruct((BT, 1), jnp.float32),
        ),
    )(fs3, ts3, y_all, bias2)

    res = res2d.reshape(BT, HID, 2)
    counts = cnt.reshape(BT, 1, 1)
    return (res, topk_indices, topk_scores, counts)


# speedup vs baseline: 3.5554x; 3.5554x over previous
import jax
import jax.numpy as jnp
from jax.experimental import pallas as pl
from jax.experimental.pallas import tpu as pltpu

HID = 1024
D2 = 2 * HID
E = 16
K = 2
BT = 4096
CHUNK = 512
N_CHUNK = BT // CHUNK
TOK_BLK = 512
N_TOK_BLK = BT // TOK_BLK
NEG_INF = float("-inf")


def _scores_kernel(x_ref, g_ref, out_ref):
    out_ref[...] = jnp.dot(x_ref[...], g_ref[...],
                           preferred_element_type=jnp.float32)


def _topk_kernel(s_ref, ti_ref, ts_ref):
    s = s_ref[...]
    iota = jax.lax.broadcasted_iota(jnp.int32, s.shape, 0)
    m1 = jnp.max(s, axis=0, keepdims=True)
    i1 = jnp.min(jnp.where(s == m1, iota, BT), axis=0, keepdims=True)
    s2 = jnp.where(iota == i1, NEG_INF, s)
    m2 = jnp.max(s2, axis=0, keepdims=True)
    i2 = jnp.min(jnp.where(s2 == m2, iota, BT), axis=0, keepdims=True)
    ti_ref[...] = jnp.concatenate([i1, i2], axis=0)
    ts_ref[...] = jnp.concatenate([m1, m2], axis=0)


def _gather_kernel(fg_ref, x_any, out_ref, sem):
    def start(j, _):
        pltpu.make_async_copy(
            x_any.at[pl.ds(fg_ref[j], 1)], out_ref.at[pl.ds(j, 1)],
            sem).start()
        return 0
    jax.lax.fori_loop(0, E * K, start, 0, unroll=True)

    def drain(j, _):
        pltpu.make_async_copy(
            x_any.at[pl.ds(fg_ref[j], 1)], out_ref.at[pl.ds(j, 1)],
            sem).wait()
        return 0
    jax.lax.fori_loop(0, E * K, drain, 0, unroll=True)


def _half_bits_to_f32(bits):
    sign = (bits & 0x8000) << 16
    expmant = (bits & 0x7FFF) << 13
    return jax.lax.bitcast_convert_type(sign | expmant, jnp.float32) * jnp.float32(
        5.192296858534828e33)


def _expert_kernel(xc_ref, w_ref, yr_ref, yi_ref):
    e = pl.program_id(0)
    wv = w_ref[...]
    wr = _half_bits_to_f32(wv & 0xFFFF).astype(jnp.bfloat16)
    wi = _half_bits_to_f32((wv >> 16) & 0xFFFF).astype(jnp.bfloat16)
    wcat = jnp.concatenate([wr, wi], axis=1)
    yy = jnp.dot(xc_ref[...], wcat, preferred_element_type=jnp.float32)
    m = E * K
    yr = yy[:m, :HID] - yy[m:, HID:]
    yi = yy[:m, HID:] + yy[m:, :HID]
    row = jax.lax.broadcasted_iota(jnp.int32, yr.shape, 0)
    own = row // K == e
    yr_ref[...] = jnp.where(own, yr, yr_ref[...])
    yi_ref[...] = jnp.where(own, yi, yi_ref[...])


def _combine_kernel(fs_ref, ts_ref, yw_ref, bias_ref, out_ref, cnt_ref):
    blk = pl.program_id(0)
    tok = jax.lax.broadcasted_iota(jnp.int32, (TOK_BLK, E * K), 0) + blk * TOK_BLK
    fs_row = fs_ref[0]
    one_hot = jnp.where(tok == fs_row, 1.0, 0.0).astype(jnp.float32)
    weight_oh = one_hot * ts_ref[0]
    out_sum = jnp.dot(weight_oh, yw_ref[...],
                      preferred_element_type=jnp.float32,
                      precision=jax.lax.Precision.HIGHEST)
    cnt = jnp.sum(one_hot, axis=1, keepdims=True)
    agg = out_sum / jnp.maximum(cnt, 1.0)
    z = agg + bias_ref[...]
    out_ref[...] = 0.5 * z * (1.0 + jax.lax.erf(z * 0.7071067811865476))
    cnt_ref[...] = cnt


@jax.jit
def kernel(x, gate_weights, experts_weight_real, act_bias):
    x_flat = x.reshape(BT, D2)

    scores = pl.pallas_call(
        _scores_kernel,
        grid=(N_CHUNK,),
        in_specs=[
            pl.BlockSpec((CHUNK, D2), lambda i: (i, 0)),
            pl.BlockSpec((D2, E), lambda i: (0, 0)),
        ],
        out_specs=pl.BlockSpec((CHUNK, E), lambda i: (i, 0)),
        out_shape=jax.ShapeDtypeStruct((BT, E), jnp.float32),
    )(x_flat, gate_weights)

    ti_t, ts_t = pl.pallas_call(
        _topk_kernel,
        out_shape=(
            jax.ShapeDtypeStruct((K, E), jnp.int32),
            jax.ShapeDtypeStruct((K, E), jnp.float32),
        ),
    )(scores)

    topk_indices = ti_t.T
    topk_scores = ts_t.T
    flat_gather = topk_indices.reshape(-1)
    flat_scatter = ti_t.reshape(-1)

    xb = pl.pallas_call(
        _gather_kernel,
        in_specs=[
            pl.BlockSpec(memory_space=pltpu.SMEM),
            pl.BlockSpec(memory_space=pl.ANY),
        ],
        out_specs=pl.BlockSpec(memory_space=pl.ANY),
        out_shape=jax.ShapeDtypeStruct((E * K, D2), jnp.float32),
        scratch_shapes=[pltpu.SemaphoreType.DMA],
    )(flat_gather, x_flat)

    xb3 = xb.reshape(E * K, HID, 2)
    xr = xb3[..., 0].astype(jnp.float16).astype(jnp.bfloat16)
    xi = xb3[..., 1].astype(jnp.float16).astype(jnp.bfloat16)
    xc = jnp.concatenate([xr, xi], axis=0)
    w_pairs = jax.lax.bitcast_convert_type(
        experts_weight_real, jnp.int32).reshape(E * HID, HID)

    yr_all, yi_all = pl.pallas_call(
        _expert_kernel,
        grid=(E,),
        in_specs=[
            pl.BlockSpec((2 * E * K, HID), lambda e: (0, 0)),
            pl.BlockSpec((HID, HID), lambda e: (e, 0)),
        ],
        out_specs=(
            pl.BlockSpec((E * K, HID), lambda e: (0, 0)),
            pl.BlockSpec((E * K, HID), lambda e: (0, 0)),
        ),
        out_shape=(
            jax.ShapeDtypeStruct((E * K, HID), jnp.float32),
            jax.ShapeDtypeStruct((E * K, HID), jnp.float32),
        ),
    )(xc, w_pairs)
    y_all = jnp.stack([yr_all, yi_all], axis=-1).reshape(E * K, D2)

    fs3 = flat_scatter.reshape(1, 1, E * K)
    ts3 = topk_scores.reshape(1, 1, E * K)
    bias2 = jnp.repeat(act_bias, 2).reshape(1, D2)

    res2d, cnt = pl.pallas_call(
        _combine_kernel,
        grid=(N_TOK_BLK,),
        in_specs=[
            pl.BlockSpec((1, 1, E * K), lambda i: (0, 0, 0)),
            pl.BlockSpec((1, 1, E * K), lambda i: (0, 0, 0)),
            pl.BlockSpec((E * K, D2), lambda i: (0, 0)),
            pl.BlockSpec((1, D2), lambda i: (0, 0)),
        ],
        out_specs=(
            pl.BlockSpec((TOK_BLK, D2), lambda i: (i, 0)),
            pl.BlockSpec((TOK_BLK, 1), lambda i: (i, 0)),
        ),
        out_shape=(
            jax.ShapeDtypeStruct((BT, D2), jnp.float32),
            jax.ShapeDtypeStruct((BT, 1), jnp.float32),
        ),
    )(fs3, ts3, y_all, bias2)

    res = res2d.reshape(BT, HID, 2)
    counts = cnt.reshape(BT, 1, 1)
    return (res, topk_indices, topk_scores, counts)


# speedup vs baseline: 3.8355x; 1.0788x over previous
import jax
import jax.numpy as jnp
from jax.experimental import pallas as pl
from jax.experimental.pallas import tpu as pltpu

HID = 1024
D2 = 2 * HID
E = 16
K = 2
BT = 4096
CHUNK = 512
N_CHUNK = BT // CHUNK
TOK_BLK = 512
N_TOK_BLK = BT // TOK_BLK
NEG_INF = float("-inf")


def _scores_kernel(x_ref, g_ref, out_ref):
    out_ref[...] = jnp.dot(x_ref[...], g_ref[...],
                           preferred_element_type=jnp.float32)


def _topk_kernel(s_ref, ti_ref, ts_ref):
    s = s_ref[...]
    iota = jax.lax.broadcasted_iota(jnp.int32, s.shape, 0)
    m1 = jnp.max(s, axis=0, keepdims=True)
    i1 = jnp.min(jnp.where(s == m1, iota, BT), axis=0, keepdims=True)
    s2 = jnp.where(iota == i1, NEG_INF, s)
    m2 = jnp.max(s2, axis=0, keepdims=True)
    i2 = jnp.min(jnp.where(s2 == m2, iota, BT), axis=0, keepdims=True)
    ti_ref[...] = jnp.concatenate([i1, i2], axis=0)
    ts_ref[...] = jnp.concatenate([m1, m2], axis=0)


def _gather_kernel(fg_ref, x_any, out_ref, sem):
    def start(j, _):
        pltpu.make_async_copy(
            x_any.at[pl.ds(fg_ref[j], 1)], out_ref.at[pl.ds(j, 1)],
            sem).start()
        return 0
    jax.lax.fori_loop(0, E * K, start, 0, unroll=True)

    def drain(j, _):
        pltpu.make_async_copy(
            x_any.at[pl.ds(fg_ref[j], 1)], out_ref.at[pl.ds(j, 1)],
            sem).wait()
        return 0
    jax.lax.fori_loop(0, E * K, drain, 0, unroll=True)


def _half_bits_to_f32(bits):
    sign = (bits & 0x8000) << 16
    expmant = (bits & 0x7FFF) << 13
    return jax.lax.bitcast_convert_type(sign | expmant, jnp.float32) * jnp.float32(
        5.192296858534828e33)


def _expert_kernel(xc_ref, w_ref, y_ref):
    e = pl.program_id(0)
    w = w_ref[...]
    yy = jnp.dot(xc_ref[...], w, preferred_element_type=jnp.float32)
    m = E * K
    a = yy[:m]
    b = yy[m:]
    lane = jax.lax.broadcasted_iota(jnp.int32, a.shape, 1)
    even = (lane % 2) == 0
    bsw = jnp.where(even, pltpu.roll(b, D2 - 1, 1), pltpu.roll(b, 1, 1))
    y = a + jnp.where(even, -bsw, bsw)
    row = jax.lax.broadcasted_iota(jnp.int32, y.shape, 0)
    own = row // K == e
    y_ref[...] = jnp.where(own, y, y_ref[...])


def _combine_kernel(fs_ref, ts_ref, yw_ref, bias_ref, out_ref, cnt_ref):
    blk = pl.program_id(0)
    tok = jax.lax.broadcasted_iota(jnp.int32, (TOK_BLK, E * K), 0) + blk * TOK_BLK
    fs_row = fs_ref[0]
    one_hot = jnp.where(tok == fs_row, 1.0, 0.0).astype(jnp.float32)
    weight_oh = one_hot * ts_ref[0]
    out_sum = jnp.dot(weight_oh, yw_ref[...],
                      preferred_element_type=jnp.float32,
                      precision=jax.lax.Precision.HIGHEST)
    cnt = jnp.sum(one_hot, axis=1, keepdims=True)
    agg = out_sum / jnp.maximum(cnt, 1.0)
    z = agg + bias_ref[...]
    out_ref[...] = 0.5 * z * (1.0 + jax.lax.erf(z * 0.7071067811865476))
    cnt_ref[...] = cnt


@jax.jit
def kernel(x, gate_weights, experts_weight_real, act_bias):
    x_flat = x.reshape(BT, D2)

    scores = pl.pallas_call(
        _scores_kernel,
        grid=(N_CHUNK,),
        in_specs=[
            pl.BlockSpec((CHUNK, D2), lambda i: (i, 0)),
            pl.BlockSpec((D2, E), lambda i: (0, 0)),
        ],
        out_specs=pl.BlockSpec((CHUNK, E), lambda i: (i, 0)),
        out_shape=jax.ShapeDtypeStruct((BT, E), jnp.float32),
    )(x_flat, gate_weights)

    ti_t, ts_t = pl.pallas_call(
        _topk_kernel,
        out_shape=(
            jax.ShapeDtypeStruct((K, E), jnp.int32),
            jax.ShapeDtypeStruct((K, E), jnp.float32),
        ),
    )(scores)

    topk_indices = ti_t.T
    topk_scores = ts_t.T
    flat_gather = topk_indices.reshape(-1)
    flat_scatter = ti_t.reshape(-1)

    xb = pl.pallas_call(
        _gather_kernel,
        in_specs=[
            pl.BlockSpec(memory_space=pltpu.SMEM),
            pl.BlockSpec(memory_space=pl.ANY),
        ],
        out_specs=pl.BlockSpec(memory_space=pl.ANY),
        out_shape=jax.ShapeDtypeStruct((E * K, D2), jnp.float32),
        scratch_shapes=[pltpu.SemaphoreType.DMA],
    )(flat_gather, x_flat)

    xb3 = xb.reshape(E * K, HID, 2)
    xr = xb3[..., 0].astype(jnp.float16).astype(jnp.bfloat16)
    xi = xb3[..., 1].astype(jnp.float16).astype(jnp.bfloat16)
    xc = jnp.concatenate([xr, xi], axis=0)
    w_bf = experts_weight_real.astype(jnp.bfloat16).reshape(E * HID, D2)

    y_all = pl.pallas_call(
        _expert_kernel,
        grid=(E,),
        in_specs=[
            pl.BlockSpec((2 * E * K, HID), lambda e: (0, 0)),
            pl.BlockSpec((HID, D2), lambda e: (e, 0)),
        ],
        out_specs=pl.BlockSpec((E * K, D2), lambda e: (0, 0)),
        out_shape=jax.ShapeDtypeStruct((E * K, D2), jnp.float32),
    )(xc, w_bf)

    fs3 = flat_scatter.reshape(1, 1, E * K)
    ts3 = topk_scores.reshape(1, 1, E * K)
    bias2 = jnp.repeat(act_bias, 2).reshape(1, D2)

    res2d, cnt = pl.pallas_call(
        _combine_kernel,
        grid=(N_TOK_BLK,),
        in_specs=[
            pl.BlockSpec((1, 1, E * K), lambda i: (0, 0, 0)),
            pl.BlockSpec((1, 1, E * K), lambda i: (0, 0, 0)),
            pl.BlockSpec((E * K, D2), lambda i: (0, 0)),
            pl.BlockSpec((1, D2), lambda i: (0, 0)),
        ],
        out_specs=(
            pl.BlockSpec((TOK_BLK, D2), lambda i: (i, 0)),
            pl.BlockSpec((TOK_BLK, 1), lambda i: (i, 0)),
        ),
        out_shape=(
            jax.ShapeDtypeStruct((BT, D2), jnp.float32),
            jax.ShapeDtypeStruct((BT, 1), jnp.float32),
        ),
    )(fs3, ts3, y_all, bias2)

    res = res2d.reshape(BT, HID, 2)
    counts = cnt.reshape(BT, 1, 1)
    return (res, topk_indices, topk_scores, counts)


# speedup vs baseline: 3.9662x; 1.0341x over previous
import jax
import jax.numpy as jnp
from jax.experimental import pallas as pl
from jax.experimental.pallas import tpu as pltpu

HID = 1024
D2 = 2 * HID
E = 16
K = 2
BT = 4096
CHUNK = 512
N_CHUNK = BT // CHUNK
TOK_BLK = 512
N_TOK_BLK = BT // TOK_BLK
NEG_INF = float("-inf")


def _scores_kernel(x_ref, g_ref, out_ref):
    out_ref[...] = jnp.dot(x_ref[...], g_ref[...],
                           preferred_element_type=jnp.float32)


def _topk_kernel(s_ref, ti_ref, ts_ref):
    s = s_ref[...]
    iota = jax.lax.broadcasted_iota(jnp.int32, s.shape, 0)
    m1 = jnp.max(s, axis=0, keepdims=True)
    i1 = jnp.min(jnp.where(s == m1, iota, BT), axis=0, keepdims=True)
    s2 = jnp.where(iota == i1, NEG_INF, s)
    m2 = jnp.max(s2, axis=0, keepdims=True)
    i2 = jnp.min(jnp.where(s2 == m2, iota, BT), axis=0, keepdims=True)
    ti_ref[...] = jnp.concatenate([i1, i2], axis=0)
    ts_ref[...] = jnp.concatenate([m1, m2], axis=0)


def _gather_kernel(fg_ref, x_any, out_ref, sem):
    def start(j, _):
        pltpu.make_async_copy(
            x_any.at[pl.ds(fg_ref[j], 1)], out_ref.at[pl.ds(j, 1)],
            sem).start()
        return 0
    jax.lax.fori_loop(0, E * K, start, 0, unroll=True)

    def drain(j, _):
        pltpu.make_async_copy(
            x_any.at[pl.ds(fg_ref[j], 1)], out_ref.at[pl.ds(j, 1)],
            sem).wait()
        return 0
    jax.lax.fori_loop(0, E * K, drain, 0, unroll=True)


def _expert_kernel(xc_ref, w_ref, y_ref):
    e = pl.program_id(0)
    w = w_ref[...]
    yy = jnp.dot(xc_ref[...], w, preferred_element_type=jnp.float32)
    m = E * K
    a = yy[:m]
    b = yy[m:]
    lane = jax.lax.broadcasted_iota(jnp.int32, a.shape, 1)
    even = (lane % 2) == 0
    bsw = jnp.where(even, pltpu.roll(b, D2 - 1, 1), pltpu.roll(b, 1, 1))
    y = a + jnp.where(even, -bsw, bsw)
    row = jax.lax.broadcasted_iota(jnp.int32, y.shape, 0)
    own = row // K == e
    y_ref[...] = jnp.where(own, y, y_ref[...])


def _combine_kernel(fs_ref, ts_ref, yw_ref, bias_ref, out_ref, cnt_ref):
    blk = pl.program_id(0)
    tok = jax.lax.broadcasted_iota(jnp.int32, (TOK_BLK, E * K), 0) + blk * TOK_BLK
    fs_row = fs_ref[0]
    one_hot = jnp.where(tok == fs_row, 1.0, 0.0).astype(jnp.float32)
    weight_oh = one_hot * ts_ref[0]
    out_sum = jnp.dot(weight_oh, yw_ref[...],
                      preferred_element_type=jnp.float32)
    cnt = jnp.sum(one_hot, axis=1, keepdims=True)
    agg = out_sum / jnp.maximum(cnt, 1.0)
    z = agg + bias_ref[...]
    out_ref[...] = 0.5 * z * (1.0 + jax.lax.erf(z * 0.7071067811865476))
    cnt_ref[...] = cnt


@jax.jit
def kernel(x, gate_weights, experts_weight_real, act_bias):
    x_flat = x.reshape(BT, D2)

    scores = pl.pallas_call(
        _scores_kernel,
        grid=(N_CHUNK,),
        in_specs=[
            pl.BlockSpec((CHUNK, D2), lambda i: (i, 0)),
            pl.BlockSpec((D2, E), lambda i: (0, 0)),
        ],
        out_specs=pl.BlockSpec((CHUNK, E), lambda i: (i, 0)),
        out_shape=jax.ShapeDtypeStruct((BT, E), jnp.float32),
    )(x_flat, gate_weights)

    ti_t, ts_t = pl.pallas_call(
        _topk_kernel,
        out_shape=(
            jax.ShapeDtypeStruct((K, E), jnp.int32),
            jax.ShapeDtypeStruct((K, E), jnp.float32),
        ),
    )(scores)

    topk_indices = ti_t.T
    topk_scores = ts_t.T
    flat_gather = topk_indices.reshape(-1)
    flat_scatter = ti_t.reshape(-1)

    xb = pl.pallas_call(
        _gather_kernel,
        in_specs=[
            pl.BlockSpec(memory_space=pltpu.SMEM),
            pl.BlockSpec(memory_space=pl.ANY),
        ],
        out_specs=pl.BlockSpec(memory_space=pl.ANY),
        out_shape=jax.ShapeDtypeStruct((E * K, D2), jnp.float32),
        scratch_shapes=[pltpu.SemaphoreType.DMA],
    )(flat_gather, x_flat)

    xb3 = xb.reshape(E * K, HID, 2)
    xr = xb3[..., 0].astype(jnp.float16).astype(jnp.bfloat16)
    xi = xb3[..., 1].astype(jnp.float16).astype(jnp.bfloat16)
    xc = jnp.concatenate([xr, xi], axis=0)
    w_bf = experts_weight_real.astype(jnp.bfloat16).reshape(E * HID, D2)

    y_all = pl.pallas_call(
        _expert_kernel,
        grid=(E,),
        in_specs=[
            pl.BlockSpec((2 * E * K, HID), lambda e: (0, 0)),
            pl.BlockSpec((HID, D2), lambda e: (e, 0)),
        ],
        out_specs=pl.BlockSpec((E * K, D2), lambda e: (0, 0)),
        out_shape=jax.ShapeDtypeStruct((E * K, D2), jnp.float32),
    )(xc, w_bf)

    fs3 = flat_scatter.reshape(1, 1, E * K)
    ts3 = topk_scores.reshape(1, 1, E * K)
    bias2 = jnp.repeat(act_bias, 2).reshape(1, D2)

    res2d, cnt = pl.pallas_call(
        _combine_kernel,
        grid=(N_TOK_BLK,),
        in_specs=[
            pl.BlockSpec((1, 1, E * K), lambda i: (0, 0, 0)),
            pl.BlockSpec((1, 1, E * K), lambda i: (0, 0, 0)),
            pl.BlockSpec((E * K, D2), lambda i: (0, 0)),
            pl.BlockSpec((1, D2), lambda i: (0, 0)),
        ],
        out_specs=(
            pl.BlockSpec((TOK_BLK, D2), lambda i: (i, 0)),
            pl.BlockSpec((TOK_BLK, 1), lambda i: (i, 0)),
        ),
        out_shape=(
            jax.ShapeDtypeStruct((BT, D2), jnp.float32),
            jax.ShapeDtypeStruct((BT, 1), jnp.float32),
        ),
    )(fs3, ts3, y_all, bias2)

    res = res2d.reshape(BT, HID, 2)
    counts = cnt.reshape(BT, 1, 1)
    return (res, topk_indices, topk_scores, counts)
